# Initial kernel scaffold; baseline (speedup 1.0000x reference)
import jax
import jax.numpy as jnp
from jax.experimental import pallas as pl


def kernel(x, edge_index, W1, b1, W2, b2):
    raise NotImplementedError("write your pallas kernel here")


# speedup vs baseline: 19.3450x; 19.3450x over previous
import functools

import jax
import jax.numpy as jnp
from jax import lax
from jax.experimental import pallas as pl
from jax.experimental.pallas import tpu as pltpu
from jax.experimental.pallas import tpu_sc as plsc

N = 10000
D = 128
NP = 10240
JUNK = NP - N
NC = 2
NS = 16
NW = NC * NS
CH = 128
K = 82
E_CAP = NW * K * CH
R = NP // NS


def _sc_degree(dst_t, ones_ch, zeros_np):
    @functools.partial(
        pl.kernel,
        mesh=plsc.VectorSubcoreMesh(core_axis_name="c", subcore_axis_name="s"),
        out_type=jax.ShapeDtypeStruct((NC, NP, D), jnp.float32),
        scratch_types=[
            pltpu.VMEM((K, CH), jnp.int32),
            pltpu.VMEM((CH, D), jnp.float32),
            pltpu.VMEM_SHARED((NP, D), jnp.float32),
        ],
    )
    def deg_kernel(dst_hbm, ones_hbm, z_hbm, out_hbm, dst_v, ones_v, acc_sh):
        c = lax.axis_index("c")
        s = lax.axis_index("s")
        wid = s * NC + c
        pltpu.sync_copy(dst_hbm.at[wid], dst_v)
        pltpu.sync_copy(ones_hbm, ones_v)
        pltpu.sync_copy(z_hbm.at[pl.ds(s * R, R)],
                        acc_sh.at[pl.ds(s * R, R)])
        plsc.subcore_barrier()

        def body(j, carry):
            pltpu.sync_copy(ones_v, acc_sh.at[dst_v.at[j]], add=True)
            return carry

        lax.fori_loop(0, K, body, 0)
        plsc.subcore_barrier()
        pltpu.sync_copy(acc_sh.at[pl.ds(s * R, R)],
                        out_hbm.at[c, pl.ds(s * R, R)])

    return deg_kernel(dst_t, ones_ch, zeros_np)


def _sc_aggregate(h, src_t, dst_t, zeros_np):
    @functools.partial(
        pl.kernel,
        mesh=plsc.VectorSubcoreMesh(core_axis_name="c", subcore_axis_name="s"),
        out_type=jax.ShapeDtypeStruct((NC, NP, D), jnp.float32),
        scratch_types=[
            pltpu.VMEM((K, CH), jnp.int32),
            pltpu.VMEM((K, CH), jnp.int32),
            pltpu.VMEM((CH, D), jnp.float32),
            pltpu.VMEM_SHARED((NP, D), jnp.float32),
            pltpu.SemaphoreType.DMA,
        ],
    )
    def agg_kernel(h_hbm, src_hbm, dst_hbm, z_hbm, out_hbm,
                   src_v, dst_v, rows_v, acc_sh, sem):
        c = lax.axis_index("c")
        s = lax.axis_index("s")
        wid = s * NC + c
        pltpu.sync_copy(src_hbm.at[wid], src_v)
        pltpu.sync_copy(dst_hbm.at[wid], dst_v)
        pltpu.sync_copy(z_hbm.at[pl.ds(s * R, R)],
                        acc_sh.at[pl.ds(s * R, R)])
        plsc.subcore_barrier()

        def body(j, carry):
            pltpu.async_copy(h_hbm.at[src_v.at[j]], rows_v, sem).wait()
            pltpu.sync_copy(rows_v, acc_sh.at[dst_v.at[j]], add=True)
            return carry

        lax.fori_loop(0, K, body, 0)
        plsc.subcore_barrier()
        pltpu.sync_copy(acc_sh.at[pl.ds(s * R, R)],
                        out_hbm.at[c, pl.ds(s * R, R)])

    return agg_kernel(h, src_t, dst_t, zeros_np)


def _tc_dense1(deg_parts, x_pad, W1):
    def body(dp_ref, x_ref, w_ref, h_ref, dinv_ref):
        degsum = dp_ref[0, :, 0:1] + dp_ref[1, :, 0:1]
        dinv = jnp.where(degsum > 0.0, lax.rsqrt(degsum), 0.0)
        h = jnp.dot(x_ref[:, :], w_ref[:, :],
                    preferred_element_type=jnp.float32)
        h_ref[:, :] = h * dinv
        dinv_ref[:, :] = dinv

    return pl.pallas_call(
        body,
        out_shape=(jax.ShapeDtypeStruct((NP, D), jnp.float32),
                   jax.ShapeDtypeStruct((NP, 1), jnp.float32)),
    )(deg_parts, x_pad, W1)


def _tc_dense2(p1, dinv, b1, W2):
    def body(p_ref, dinv_ref, b_ref, w_ref, out_ref):
        a = (p_ref[0] + p_ref[1]) * dinv_ref[:, :] + b_ref[:, :]
        a = jnp.maximum(a, 0.0)
        out_ref[:, :] = jnp.dot(a, w_ref[:, :],
                                preferred_element_type=jnp.float32) * dinv_ref[:, :]

    return pl.pallas_call(
        body,
        out_shape=jax.ShapeDtypeStruct((NP, D), jnp.float32),
    )(p1, dinv, b1, W2)


def _tc_dense3(p2, dinv, b2):
    def body(p_ref, dinv_ref, b_ref, out_ref):
        out_ref[:, :] = (p_ref[0] + p_ref[1]) * dinv_ref[:, :] + b_ref[:, :]

    return pl.pallas_call(
        body,
        out_shape=jax.ShapeDtypeStruct((NP, D), jnp.float32),
    )(p2, dinv, b2)


def kernel(x, edge_index, W1, b1, W2, b2):
    ei = edge_index.astype(jnp.int32)
    loop = jnp.arange(N, dtype=jnp.int32)
    src = jnp.concatenate([ei[0], loop])
    dst = jnp.concatenate([ei[1], loop])
    npad = E_CAP - src.shape[0]
    pad_idx = N + (jnp.arange(npad, dtype=jnp.int32) % JUNK)
    src_t = jnp.concatenate([src, pad_idx]).reshape(NW, K, CH)
    dst_t = jnp.concatenate([dst, pad_idx]).reshape(NW, K, CH)
    x_pad = jnp.zeros((NP, D), jnp.float32).at[:N].set(x)
    zeros_np = jnp.zeros((NP, D), jnp.float32)

    ones_ch = jnp.ones((CH, D), jnp.float32)
    deg_parts = _sc_degree(dst_t, ones_ch, zeros_np)
    h1, dinv = _tc_dense1(deg_parts, x_pad, W1)
    p1 = _sc_aggregate(h1, src_t, dst_t, zeros_np)
    h2 = _tc_dense2(p1, dinv, b1.reshape(1, D), W2)
    p2 = _sc_aggregate(h2, src_t, dst_t, zeros_np)
    out = _tc_dense3(p2, dinv, b2.reshape(1, D))
    return out[:N]


# speedup vs baseline: 23.1281x; 1.1956x over previous
import functools

import jax
import jax.numpy as jnp
from jax import lax
from jax.experimental import pallas as pl
from jax.experimental.pallas import tpu as pltpu
from jax.experimental.pallas import tpu_sc as plsc

N = 10000
D = 128
NP = 10240
JUNK = NP - N
NC = 2
NS = 16
NW = NC * NS
CH = 128
K = 84
KH = K // 2
E_CAP = NW * K * CH
R = NP // NS


def _sc_degree(dst_t, ones_ch, zeros_np):
    @functools.partial(
        pl.kernel,
        mesh=plsc.VectorSubcoreMesh(core_axis_name="c", subcore_axis_name="s"),
        out_type=jax.ShapeDtypeStruct((NC, NP, D), jnp.float32),
        scratch_types=[
            pltpu.VMEM((K, CH), jnp.int32),
            pltpu.VMEM((CH, D), jnp.float32),
            pltpu.VMEM_SHARED((NP, D), jnp.float32),
        ],
    )
    def deg_kernel(dst_hbm, ones_hbm, z_hbm, out_hbm, dst_v, ones_v, acc_sh):
        c = lax.axis_index("c")
        s = lax.axis_index("s")
        wid = s * NC + c
        pltpu.sync_copy(dst_hbm.at[wid], dst_v)
        pltpu.sync_copy(ones_hbm, ones_v)
        pltpu.sync_copy(z_hbm.at[pl.ds(s * R, R)],
                        acc_sh.at[pl.ds(s * R, R)])
        plsc.subcore_barrier()

        def body(j, carry):
            pltpu.sync_copy(ones_v, acc_sh.at[dst_v.at[j]], add=True)
            return carry

        lax.fori_loop(0, K, body, 0)
        plsc.subcore_barrier()
        pltpu.sync_copy(acc_sh.at[pl.ds(s * R, R)],
                        out_hbm.at[c, pl.ds(s * R, R)])

    return deg_kernel(dst_t, ones_ch, zeros_np)


def _sc_aggregate(h, src_t, dst_t, zeros_np):
    @functools.partial(
        pl.kernel,
        mesh=plsc.VectorSubcoreMesh(core_axis_name="c", subcore_axis_name="s"),
        out_type=jax.ShapeDtypeStruct((NC, NP, D), jnp.float32),
        scratch_types=[
            pltpu.VMEM((KH, CH), jnp.int32),
            pltpu.VMEM((KH, CH), jnp.int32),
            pltpu.VMEM((CH, D), jnp.float32),
            pltpu.VMEM((CH, D), jnp.float32),
            pltpu.VMEM_SHARED((NP, D), jnp.float32),
            pltpu.SemaphoreType.DMA,
        ],
    )
    def agg_kernel(h_hbm, src_hbm, dst_hbm, z_hbm, out_hbm,
                   src_v, dst_v, rows0_v, rows1_v, acc_sh, sem0):
        c = lax.axis_index("c")
        s = lax.axis_index("s")
        wid = s * NC + c
        pltpu.sync_copy(z_hbm.at[pl.ds(s * R, R)],
                        acc_sh.at[pl.ds(s * R, R)])
        plsc.subcore_barrier()

        def half(h):
            pltpu.sync_copy(src_hbm.at[wid, h], src_v)
            pltpu.sync_copy(dst_hbm.at[wid, h], dst_v)

            def body(i, carry):
                j0 = 2 * i
                d0 = pltpu.async_copy(h_hbm.at[src_v.at[j0]], rows0_v, sem0)
                d1 = pltpu.async_copy(h_hbm.at[src_v.at[j0 + 1]], rows1_v,
                                      sem0)
                d0.wait()
                pltpu.sync_copy(rows0_v, acc_sh.at[dst_v.at[j0]], add=True)
                d1.wait()
                pltpu.sync_copy(rows1_v, acc_sh.at[dst_v.at[j0 + 1]],
                                add=True)
                return carry

            lax.fori_loop(0, KH // 2, body, 0)

        half(0)
        half(1)
        plsc.subcore_barrier()
        pltpu.sync_copy(acc_sh.at[pl.ds(s * R, R)],
                        out_hbm.at[c, pl.ds(s * R, R)])

    return agg_kernel(h, src_t, dst_t, zeros_np)


def _tc_dense1(deg_parts, x_pad, W1):
    def body(dp_ref, x_ref, w_ref, h_ref, dinv_ref):
        degsum = dp_ref[0, :, 0:1] + dp_ref[1, :, 0:1]
        dinv = jnp.where(degsum > 0.0, lax.rsqrt(degsum), 0.0)
        h = jnp.dot(x_ref[:, :], w_ref[:, :],
                    preferred_element_type=jnp.float32)
        h_ref[:, :] = h * dinv
        dinv_ref[:, :] = dinv

    return pl.pallas_call(
        body,
        out_shape=(jax.ShapeDtypeStruct((NP, D), jnp.float32),
                   jax.ShapeDtypeStruct((NP, 1), jnp.float32)),
    )(deg_parts, x_pad, W1)


def _tc_dense2(p1, dinv, b1, W2):
    def body(p_ref, dinv_ref, b_ref, w_ref, out_ref):
        a = (p_ref[0] + p_ref[1]) * dinv_ref[:, :] + b_ref[:, :]
        a = jnp.maximum(a, 0.0)
        out_ref[:, :] = jnp.dot(a, w_ref[:, :],
                                preferred_element_type=jnp.float32) * dinv_ref[:, :]

    return pl.pallas_call(
        body,
        out_shape=jax.ShapeDtypeStruct((NP, D), jnp.float32),
    )(p1, dinv, b1, W2)


def _tc_dense3(p2, dinv, b2):
    def body(p_ref, dinv_ref, b_ref, out_ref):
        out_ref[:, :] = (p_ref[0] + p_ref[1]) * dinv_ref[:, :] + b_ref[:, :]

    return pl.pallas_call(
        body,
        out_shape=jax.ShapeDtypeStruct((NP, D), jnp.float32),
    )(p2, dinv, b2)


def kernel(x, edge_index, W1, b1, W2, b2):
    ei = edge_index.astype(jnp.int32)
    loop = jnp.arange(N, dtype=jnp.int32)
    src = jnp.concatenate([ei[0], loop])
    dst = jnp.concatenate([ei[1], loop])
    npad = E_CAP - src.shape[0]
    pad_idx = N + (jnp.arange(npad, dtype=jnp.int32) % JUNK)
    src_t = jnp.concatenate([src, pad_idx]).reshape(NW, 2, KH, CH)
    dst_t = jnp.concatenate([dst, pad_idx]).reshape(NW, 2, KH, CH)
    x_pad = jnp.zeros((NP, D), jnp.float32).at[:N].set(x)
    zeros_np = jnp.zeros((NP, D), jnp.float32)

    ones_ch = jnp.ones((CH, D), jnp.float32)
    deg_parts = _sc_degree(dst_t.reshape(NW, K, CH), ones_ch, zeros_np)
    h1, dinv = _tc_dense1(deg_parts, x_pad, W1)
    p1 = _sc_aggregate(h1, src_t, dst_t, zeros_np)
    h2 = _tc_dense2(p1, dinv, b1.reshape(1, D), W2)
    p2 = _sc_aggregate(h2, src_t, dst_t, zeros_np)
    out = _tc_dense3(p2, dinv, b2.reshape(1, D))
    return out[:N]


# speedup vs baseline: 26.7974x; 1.1587x over previous
import functools

import jax
import jax.numpy as jnp
from jax import lax
from jax.experimental import pallas as pl
from jax.experimental.pallas import tpu as pltpu
from jax.experimental.pallas import tpu_sc as plsc

N = 10000
D = 128
NP = 10112
JUNK = NP - N
NC = 2
NS = 16
NW = NC * NS
CH = 112
K = 96
NQ = 4
KH = K // NQ
E_CAP = NW * K * CH
R = NP // NS
WD = 128


def _sc_degree(dst_t, ones_ch, zeros_w):
    @functools.partial(
        pl.kernel,
        mesh=plsc.VectorSubcoreMesh(core_axis_name="c", subcore_axis_name="s"),
        out_type=jax.ShapeDtypeStruct((NC, NP, WD), jnp.float32),
        scratch_types=[
            pltpu.VMEM((K, CH), jnp.int32),
            pltpu.VMEM((CH, WD), jnp.float32),
            pltpu.VMEM_SHARED((NP, WD), jnp.float32),
        ],
    )
    def deg_kernel(dst_hbm, ones_hbm, z_hbm, out_hbm, dst_v, ones_v, acc_sh):
        c = lax.axis_index("c")
        s = lax.axis_index("s")
        wid = s * NC + c
        pltpu.sync_copy(dst_hbm.at[wid], dst_v)
        pltpu.sync_copy(ones_hbm, ones_v)
        pltpu.sync_copy(z_hbm.at[pl.ds(s * R, R)],
                        acc_sh.at[pl.ds(s * R, R)])
        plsc.subcore_barrier()

        def body(j, carry):
            pltpu.sync_copy(ones_v, acc_sh.at[dst_v.at[j]], add=True)
            return carry

        lax.fori_loop(0, K, body, 0)
        plsc.subcore_barrier()
        pltpu.sync_copy(acc_sh.at[pl.ds(s * R, R)],
                        out_hbm.at[c, pl.ds(s * R, R)])

    return deg_kernel(dst_t, ones_ch, zeros_w)


def _sc_aggregate(h, src_t, dst_t, zeros_np):
    @functools.partial(
        pl.kernel,
        mesh=plsc.VectorSubcoreMesh(core_axis_name="c", subcore_axis_name="s"),
        out_type=jax.ShapeDtypeStruct((NC, NP, D), jnp.float32),
        scratch_types=[
            pltpu.VMEM((KH, CH), jnp.int32),
            pltpu.VMEM((KH, CH), jnp.int32),
            pltpu.VMEM((CH, D), jnp.float32),
            pltpu.VMEM((CH, D), jnp.float32),
            pltpu.VMEM((CH, D), jnp.float32),
            pltpu.VMEM_SHARED((NP, D), jnp.float32),
            pltpu.SemaphoreType.DMA,
            pltpu.SemaphoreType.DMA,
            pltpu.SemaphoreType.DMA,
        ],
    )
    def agg_kernel(h_hbm, src_hbm, dst_hbm, z_hbm, out_hbm,
                   src_v, dst_v, b0, b1, b2, acc_sh, sem0, sem1, sem2):
        c = lax.axis_index("c")
        s = lax.axis_index("s")
        wid = s * NC + c
        pltpu.sync_copy(z_hbm.at[pl.ds(s * R, R)],
                        acc_sh.at[pl.ds(s * R, R)])
        plsc.subcore_barrier()

        def gather(t, buf, sem):
            return pltpu.async_copy(h_hbm.at[src_v.at[t]], buf, sem)

        def quarter(h):
            pltpu.sync_copy(src_hbm.at[wid, h], src_v)
            pltpu.sync_copy(dst_hbm.at[wid, h], dst_v)
            gather(0, b0, sem0)
            gather(1, b1, sem1)

            def step(t, buf, sem, nxt_buf, nxt_sem):
                pltpu.make_async_copy(h_hbm.at[src_v.at[t]], buf,
                                      sem).wait()

                @pl.when(t + 2 < KH)
                def _():
                    gather(t + 2, nxt_buf, nxt_sem)

                pltpu.sync_copy(buf, acc_sh.at[dst_v.at[t]], add=True)

            def body(i, carry):
                t = 3 * i
                step(t, b0, sem0, b2, sem2)
                step(t + 1, b1, sem1, b0, sem0)
                step(t + 2, b2, sem2, b1, sem1)
                return carry

            lax.fori_loop(0, KH // 3, body, 0)

        for h in range(NQ):
            quarter(h)
        plsc.subcore_barrier()
        pltpu.sync_copy(acc_sh.at[pl.ds(s * R, R)],
                        out_hbm.at[c, pl.ds(s * R, R)])

    return agg_kernel(h, src_t, dst_t, zeros_np)


def _tc_dense1(deg_parts, x_pad, W1):
    def body(dp_ref, x_ref, w_ref, h_ref, dinv_ref):
        degsum = dp_ref[0, :, 0:1] + dp_ref[1, :, 0:1]
        dinv = jnp.where(degsum > 0.0, lax.rsqrt(degsum), 0.0)
        h = jnp.dot(x_ref[:, :], w_ref[:, :],
                    preferred_element_type=jnp.float32)
        h_ref[:, :] = h * dinv
        dinv_ref[:, :] = dinv

    return pl.pallas_call(
        body,
        out_shape=(jax.ShapeDtypeStruct((NP, D), jnp.float32),
                   jax.ShapeDtypeStruct((NP, 1), jnp.float32)),
    )(deg_parts, x_pad, W1)


def _tc_dense2(p1, dinv, b1, W2):
    def body(p_ref, dinv_ref, b_ref, w_ref, out_ref):
        a = (p_ref[0] + p_ref[1]) * dinv_ref[:, :] + b_ref[:, :]
        a = jnp.maximum(a, 0.0)
        out_ref[:, :] = jnp.dot(a, w_ref[:, :],
                                preferred_element_type=jnp.float32) * dinv_ref[:, :]

    return pl.pallas_call(
        body,
        out_shape=jax.ShapeDtypeStruct((NP, D), jnp.float32),
    )(p1, dinv, b1, W2)


def _tc_dense3(p2, dinv, b2):
    def body(p_ref, dinv_ref, b_ref, out_ref):
        out_ref[:, :] = (p_ref[0] + p_ref[1]) * dinv_ref[:, :] + b_ref[:, :]

    return pl.pallas_call(
        body,
        out_shape=jax.ShapeDtypeStruct((NP, D), jnp.float32),
    )(p2, dinv, b2)


def kernel(x, edge_index, W1, b1, W2, b2):
    ei = edge_index.astype(jnp.int32)
    loop = jnp.arange(N, dtype=jnp.int32)
    src = jnp.concatenate([ei[0], loop])
    dst = jnp.concatenate([ei[1], loop])
    npad = E_CAP - src.shape[0]
    pad_idx = N + (jnp.arange(npad, dtype=jnp.int32) % JUNK)
    src_t = jnp.concatenate([src, pad_idx]).reshape(NW, NQ, KH, CH)
    dst_t = jnp.concatenate([dst, pad_idx]).reshape(NW, NQ, KH, CH)
    x_pad = jnp.zeros((NP, D), jnp.float32).at[:N].set(x)
    zeros_np = jnp.zeros((NP, D), jnp.float32)

    ones_ch = jnp.ones((CH, WD), jnp.float32)
    zeros_w = jnp.zeros((NP, WD), jnp.float32)
    deg_parts = _sc_degree(dst_t.reshape(NW, K, CH), ones_ch, zeros_w)
    h1, dinv = _tc_dense1(deg_parts, x_pad, W1)
    p1 = _sc_aggregate(h1, src_t, dst_t, zeros_np)
    h2 = _tc_dense2(p1, dinv, b1.reshape(1, D), W2)
    p2 = _sc_aggregate(h2, src_t, dst_t, zeros_np)
    out = _tc_dense3(p2, dinv, b2.reshape(1, D))
    return out[:N]


# speedup vs baseline: 26.8161x; 1.0007x over previous
import functools

import jax
import jax.numpy as jnp
from jax import lax
from jax.experimental import pallas as pl
from jax.experimental.pallas import tpu as pltpu
from jax.experimental.pallas import tpu_sc as plsc

N = 10000
D = 128
NP = 10112
JUNK = NP - N
NC = 2
NS = 16
NW = NC * NS
CH = 112
K = 96
NQ = 4
KH = K // NQ
E_CAP = NW * K * CH
R = NP // NS
WD = 128


def _sc_degree(dst_t, ones_ch, zeros_w):
    @functools.partial(
        pl.kernel,
        mesh=plsc.VectorSubcoreMesh(core_axis_name="c", subcore_axis_name="s"),
        out_type=jax.ShapeDtypeStruct((NC, NP, WD), jnp.float32),
        scratch_types=[
            pltpu.VMEM((K, CH), jnp.int32),
            pltpu.VMEM((CH, WD), jnp.float32),
            pltpu.VMEM_SHARED((NP, WD), jnp.float32),
        ],
    )
    def deg_kernel(dst_hbm, ones_hbm, z_hbm, out_hbm, dst_v, ones_v, acc_sh):
        c = lax.axis_index("c")
        s = lax.axis_index("s")
        wid = s * NC + c
        pltpu.sync_copy(dst_hbm.at[wid], dst_v)
        pltpu.sync_copy(ones_hbm, ones_v)
        pltpu.sync_copy(z_hbm.at[pl.ds(s * R, R)],
                        acc_sh.at[pl.ds(s * R, R)])
        plsc.subcore_barrier()

        def body(j, carry):
            pltpu.sync_copy(ones_v, acc_sh.at[dst_v.at[j]], add=True)
            return carry

        lax.fori_loop(0, K, body, 0)
        plsc.subcore_barrier()
        pltpu.sync_copy(acc_sh.at[pl.ds(s * R, R)],
                        out_hbm.at[c, pl.ds(s * R, R)])

    return deg_kernel(dst_t, ones_ch, zeros_w)


def _sc_aggregate(h, src_t, dst_t, zeros_np):
    @functools.partial(
        pl.kernel,
        mesh=plsc.VectorSubcoreMesh(core_axis_name="c", subcore_axis_name="s"),
        out_type=jax.ShapeDtypeStruct((NC, NP, D), jnp.float32),
        scratch_types=[
            pltpu.VMEM((KH, CH), jnp.int32),
            pltpu.VMEM((KH, CH), jnp.int32),
            pltpu.VMEM((CH, D), jnp.float32),
            pltpu.VMEM((CH, D), jnp.float32),
            pltpu.VMEM((CH, D), jnp.float32),
            pltpu.VMEM_SHARED((NP, D), jnp.float32),
            pltpu.SemaphoreType.DMA,
            pltpu.SemaphoreType.DMA,
            pltpu.SemaphoreType.DMA,
        ],
    )
    def agg_kernel(h_hbm, src_hbm, dst_hbm, z_hbm, out_hbm,
                   src_v, dst_v, b0, b1, b2, acc_sh, sem0, sem1, sem2):
        c = lax.axis_index("c")
        s = lax.axis_index("s")
        wid = s * NC + c
        pltpu.sync_copy(z_hbm.at[pl.ds(s * R, R)],
                        acc_sh.at[pl.ds(s * R, R)])
        plsc.subcore_barrier()

        def gather(t, buf, sem):
            return pltpu.async_copy(h_hbm.at[src_v.at[t]], buf, sem)

        def quarter(h):
            pltpu.sync_copy(src_hbm.at[wid, h], src_v)
            pltpu.sync_copy(dst_hbm.at[wid, h], dst_v)
            gather(0, b0, sem0)
            gather(1, b1, sem1)

            def step(t, buf, sem, nxt_buf, nxt_sem):
                pltpu.make_async_copy(h_hbm.at[src_v.at[t]], buf,
                                      sem).wait()

                @pl.when(t + 2 < KH)
                def _():
                    gather(t + 2, nxt_buf, nxt_sem)

                pltpu.sync_copy(buf, acc_sh.at[dst_v.at[t]], add=True)

            def body(i, carry):
                t = 3 * i
                step(t, b0, sem0, b2, sem2)
                step(t + 1, b1, sem1, b0, sem0)
                step(t + 2, b2, sem2, b1, sem1)
                return carry

            lax.fori_loop(0, KH // 3, body, 0)

        for h in range(NQ):
            quarter(h)
        plsc.subcore_barrier()
        pltpu.sync_copy(acc_sh.at[pl.ds(s * R, R)],
                        out_hbm.at[c, pl.ds(s * R, R)])

    return agg_kernel(h, src_t, dst_t, zeros_np)


def _tc_matmul(x_pad, W1):
    def body(x_ref, w_ref, h_ref):
        h_ref[:, :] = jnp.dot(x_ref[:, :], w_ref[:, :],
                              preferred_element_type=jnp.float32)

    return pl.pallas_call(
        body,
        out_shape=jax.ShapeDtypeStruct((NP, D), jnp.float32),
    )(x_pad, W1)


def _tc_scale1(deg_parts, h):
    def body(dp_ref, h_ref, h1_ref, dinv_ref):
        degsum = dp_ref[0, :, 0:1] + dp_ref[1, :, 0:1]
        dinv = jnp.where(degsum > 0.0, lax.rsqrt(degsum), 0.0)
        h1_ref[:, :] = h_ref[:, :] * dinv
        dinv_ref[:, :] = dinv

    return pl.pallas_call(
        body,
        out_shape=(jax.ShapeDtypeStruct((NP, D), jnp.float32),
                   jax.ShapeDtypeStruct((NP, 1), jnp.float32)),
    )(deg_parts, h)


def _tc_dense2(p1, dinv, b1, W2):
    def body(p_ref, dinv_ref, b_ref, w_ref, out_ref):
        a = (p_ref[0] + p_ref[1]) * dinv_ref[:, :] + b_ref[:, :]
        a = jnp.maximum(a, 0.0)
        out_ref[:, :] = jnp.dot(a, w_ref[:, :],
                                preferred_element_type=jnp.float32) * dinv_ref[:, :]

    return pl.pallas_call(
        body,
        out_shape=jax.ShapeDtypeStruct((NP, D), jnp.float32),
    )(p1, dinv, b1, W2)


def _tc_dense3(p2, dinv, b2):
    def body(p_ref, dinv_ref, b_ref, out_ref):
        out_ref[:, :] = (p_ref[0] + p_ref[1]) * dinv_ref[:, :] + b_ref[:, :]

    return pl.pallas_call(
        body,
        out_shape=jax.ShapeDtypeStruct((NP, D), jnp.float32),
    )(p2, dinv, b2)


def kernel(x, edge_index, W1, b1, W2, b2):
    ei = edge_index.astype(jnp.int32)
    loop = jnp.arange(N, dtype=jnp.int32)
    src = jnp.concatenate([ei[0], loop])
    dst = jnp.concatenate([ei[1], loop])
    npad = E_CAP - src.shape[0]
    pad_idx = N + (jnp.arange(npad, dtype=jnp.int32) % JUNK)
    src_t = jnp.concatenate([src, pad_idx]).reshape(NW, NQ, KH, CH)
    dst_t = jnp.concatenate([dst, pad_idx]).reshape(NW, NQ, KH, CH)
    x_pad = jnp.zeros((NP, D), jnp.float32).at[:N].set(x)
    zeros_np = jnp.zeros((NP, D), jnp.float32)

    ones_ch = jnp.ones((CH, WD), jnp.float32)
    zeros_w = jnp.zeros((NP, WD), jnp.float32)
    deg_parts = _sc_degree(dst_t.reshape(NW, K, CH), ones_ch, zeros_w)
    h = _tc_matmul(x_pad, W1)
    h1, dinv = _tc_scale1(deg_parts, h)
    p1 = _sc_aggregate(h1, src_t, dst_t, zeros_np)
    h2 = _tc_dense2(p1, dinv, b1.reshape(1, D), W2)
    p2 = _sc_aggregate(h2, src_t, dst_t, zeros_np)
    out = _tc_dense3(p2, dinv, b2.reshape(1, D))
    return out[:N]


# speedup vs baseline: 27.3503x; 1.0199x over previous
import functools

import jax
import jax.numpy as jnp
from jax import lax
from jax.experimental import pallas as pl
from jax.experimental.pallas import tpu as pltpu
from jax.experimental.pallas import tpu_sc as plsc

N = 10000
D = 128
NP = 10112
JUNK = NP - N
NC = 2
NS = 16
NW = NC * NS
CH = 108
K = 96
NQ = 4
KH = K // NQ
E_CAP = NW * K * CH
R = NP // NS
WD = 128


def _sc_degree(dst_t, ones_ch, zeros_w):
    @functools.partial(
        pl.kernel,
        mesh=plsc.VectorSubcoreMesh(core_axis_name="c", subcore_axis_name="s"),
        out_type=jax.ShapeDtypeStruct((NC, NP, WD), jnp.float32),
        scratch_types=[
            pltpu.VMEM((K, CH), jnp.int32),
            pltpu.VMEM((CH, WD), jnp.float32),
            pltpu.VMEM_SHARED((NP, WD), jnp.float32),
        ],
    )
    def deg_kernel(dst_hbm, ones_hbm, z_hbm, out_hbm, dst_v, ones_v, acc_sh):
        c = lax.axis_index("c")
        s = lax.axis_index("s")
        wid = s * NC + c
        pltpu.sync_copy(dst_hbm.at[wid], dst_v)
        pltpu.sync_copy(ones_hbm, ones_v)
        pltpu.sync_copy(z_hbm.at[pl.ds(s * R, R)],
                        acc_sh.at[pl.ds(s * R, R)])
        plsc.subcore_barrier()

        def body(j, carry):
            pltpu.sync_copy(ones_v, acc_sh.at[dst_v.at[j]], add=True)
            return carry

        lax.fori_loop(0, K, body, 0)
        plsc.subcore_barrier()
        pltpu.sync_copy(acc_sh.at[pl.ds(s * R, R)],
                        out_hbm.at[c, pl.ds(s * R, R)])

    return deg_kernel(dst_t, ones_ch, zeros_w)


def _sc_aggregate(h, src_t, dst_t, zeros_np):
    @functools.partial(
        pl.kernel,
        mesh=plsc.VectorSubcoreMesh(core_axis_name="c", subcore_axis_name="s"),
        out_type=jax.ShapeDtypeStruct((NC, NP, D), jnp.float32),
        scratch_types=[
            pltpu.VMEM((KH, CH), jnp.int32),
            pltpu.VMEM((KH, CH), jnp.int32),
            pltpu.VMEM((CH, D), jnp.float32),
            pltpu.VMEM((CH, D), jnp.float32),
            pltpu.VMEM((CH, D), jnp.float32),
            pltpu.VMEM_SHARED((NP, D), jnp.float32),
            pltpu.SemaphoreType.DMA,
            pltpu.SemaphoreType.DMA,
            pltpu.SemaphoreType.DMA,
        ],
    )
    def agg_kernel(h_hbm, src_hbm, dst_hbm, z_hbm, out_hbm,
                   src_v, dst_v, b0, b1, b2, acc_sh, sem0, sem1, sem2):
        c = lax.axis_index("c")
        s = lax.axis_index("s")
        wid = s * NC + c
        pltpu.sync_copy(z_hbm.at[pl.ds(s * R, R)],
                        acc_sh.at[pl.ds(s * R, R)])
        plsc.subcore_barrier()

        def gather(t, buf, sem):
            return pltpu.async_copy(h_hbm.at[src_v.at[t]], buf, sem)

        def quarter(h):
            pltpu.sync_copy(src_hbm.at[wid, h], src_v)
            pltpu.sync_copy(dst_hbm.at[wid, h], dst_v)
            gather(0, b0, sem0)
            gather(1, b1, sem1)

            def step(t, buf, sem, nxt_buf, nxt_sem):
                pltpu.make_async_copy(h_hbm.at[src_v.at[t]], buf,
                                      sem).wait()

                @pl.when(t + 2 < KH)
                def _():
                    gather(t + 2, nxt_buf, nxt_sem)

                pltpu.sync_copy(buf, acc_sh.at[dst_v.at[t]], add=True)

            def body(i, carry):
                t = 3 * i
                step(t, b0, sem0, b2, sem2)
                step(t + 1, b1, sem1, b0, sem0)
                step(t + 2, b2, sem2, b1, sem1)
                return carry

            lax.fori_loop(0, KH // 3, body, 0)

        for h in range(NQ):
            quarter(h)
        plsc.subcore_barrier()
        pltpu.sync_copy(acc_sh.at[pl.ds(s * R, R)],
                        out_hbm.at[c, pl.ds(s * R, R)])

    return agg_kernel(h, src_t, dst_t, zeros_np)


def _tc_matmul(x_pad, W1):
    def body(x_ref, w_ref, h_ref):
        h_ref[:, :] = jnp.dot(x_ref[:, :], w_ref[:, :],
                              preferred_element_type=jnp.float32)

    return pl.pallas_call(
        body,
        out_shape=jax.ShapeDtypeStruct((NP, D), jnp.float32),
    )(x_pad, W1)


def _tc_scale1(deg_parts, h):
    def body(dp_ref, h_ref, h1_ref, dinv_ref):
        degsum = dp_ref[0, :, 0:1] + dp_ref[1, :, 0:1]
        dinv = jnp.where(degsum > 0.0, lax.rsqrt(degsum), 0.0)
        h1_ref[:, :] = h_ref[:, :] * dinv
        dinv_ref[:, :] = dinv

    return pl.pallas_call(
        body,
        out_shape=(jax.ShapeDtypeStruct((NP, D), jnp.float32),
                   jax.ShapeDtypeStruct((NP, 1), jnp.float32)),
    )(deg_parts, h)


def _tc_dense2(p1, dinv, b1, W2):
    def body(p_ref, dinv_ref, b_ref, w_ref, out_ref):
        a = (p_ref[0] + p_ref[1]) * dinv_ref[:, :] + b_ref[:, :]
        a = jnp.maximum(a, 0.0)
        out_ref[:, :] = jnp.dot(a, w_ref[:, :],
                                preferred_element_type=jnp.float32) * dinv_ref[:, :]

    return pl.pallas_call(
        body,
        out_shape=jax.ShapeDtypeStruct((NP, D), jnp.float32),
    )(p1, dinv, b1, W2)


def _tc_dense3(p2, dinv, b2):
    def body(p_ref, dinv_ref, b_ref, out_ref):
        out_ref[:, :] = (p_ref[0] + p_ref[1]) * dinv_ref[:, :] + b_ref[:, :]

    return pl.pallas_call(
        body,
        out_shape=jax.ShapeDtypeStruct((NP, D), jnp.float32),
    )(p2, dinv, b2)


def kernel(x, edge_index, W1, b1, W2, b2):
    ei = edge_index.astype(jnp.int32)
    loop = jnp.arange(N, dtype=jnp.int32)
    src = jnp.concatenate([ei[0], loop])
    dst = jnp.concatenate([ei[1], loop])
    npad = E_CAP - src.shape[0]
    pad_idx = N + (jnp.arange(npad, dtype=jnp.int32) % JUNK)
    src_t = jnp.concatenate([src, pad_idx]).reshape(NW, NQ, KH, CH)
    dst_t = jnp.concatenate([dst, pad_idx]).reshape(NW, NQ, KH, CH)
    x_pad = jnp.zeros((NP, D), jnp.float32).at[:N].set(x)
    zeros_np = jnp.zeros((NP, D), jnp.float32)

    ones_ch = jnp.ones((CH, WD), jnp.float32)
    zeros_w = jnp.zeros((NP, WD), jnp.float32)
    deg_parts = _sc_degree(dst_t.reshape(NW, K, CH), ones_ch, zeros_w)
    h = _tc_matmul(x_pad, W1)
    h1, dinv = _tc_scale1(deg_parts, h)
    p1 = _sc_aggregate(h1, src_t, dst_t, zeros_np)
    h2 = _tc_dense2(p1, dinv, b1.reshape(1, D), W2)
    p2 = _sc_aggregate(h2, src_t, dst_t, zeros_np)
    out = _tc_dense3(p2, dinv, b2.reshape(1, D))
    return out[:N]
